# baseline (speedup 1.0000x reference)
import functools

import jax
import jax.numpy as jnp
from jax import lax
from jax.experimental import pallas as pl
from jax.experimental.pallas import tpu as pltpu
from jax.experimental.pallas import tpu_sc as plsc

B = 16384
TB = 2 * B
D = 64
W = 2 * D
L = 16
NC, NS = 2, 16
NW = NC * NS
BPW = TB // NW
C = 128
NCHUNK = BPW // C


def _tc_transpose_concat(ent_re_t, ent_im_t, rel_re_t, rel_im_t):
    N = ent_re_t.shape[1]
    EB = 8192

    def body(a_ref, b_ref, c_ref, d_ref, e_ref, r_ref):
        e_ref[:, :D] = a_ref[...].T
        e_ref[:, D:] = b_ref[...].T
        r_ref[:, :D] = c_ref[...].T
        r_ref[:, D:] = d_ref[...].T

    in_spec = pl.BlockSpec((D, EB), lambda i: (0, i))
    out_spec = pl.BlockSpec((EB, W), lambda i: (i, 0))
    return pl.pallas_call(
        body,
        grid=(pl.cdiv(N, EB),),
        in_specs=[in_spec] * 4,
        out_specs=[out_spec, out_spec],
        out_shape=[jax.ShapeDtypeStruct((N, W), jnp.float32)] * 2,
    )(ent_re_t, ent_im_t, rel_re_t, rel_im_t)


def _sc_partials(ent, rel, idx):
    mesh = plsc.VectorSubcoreMesh(core_axis_name="c", subcore_axis_name="s")

    @functools.partial(
        pl.kernel,
        mesh=mesh,
        out_type=jax.ShapeDtypeStruct((TB, L), jnp.float32),
        compiler_params=pltpu.CompilerParams(use_tc_tiling_on_sc=True),
        scratch_types=[
            pltpu.VMEM((2, C), jnp.int32),
            pltpu.VMEM((2, C), jnp.int32),
            pltpu.VMEM((2, C), jnp.int32),
            pltpu.VMEM((C, W), jnp.float32),
            pltpu.VMEM((C, W), jnp.float32),
            pltpu.VMEM((C, W), jnp.float32),
            pltpu.VMEM((C, W), jnp.float32),
            pltpu.VMEM((C, W), jnp.float32),
            pltpu.VMEM((C, W), jnp.float32),
            pltpu.VMEM((C, L), jnp.float32),
            pltpu.SemaphoreType.DMA,
            pltpu.SemaphoreType.DMA,
        ],
    )
    def kern(ent_h, rel_h, idx_h, out_h,
             hi_v, ri_v, ti_v, eh0, et0, er0, eh1, et1, er1,
             acc_v, sem0, sem1):
        wid = lax.axis_index("s") * NC + lax.axis_index("c")
        base = wid * BPW
        bufs = ((eh0, et0, er0, sem0), (eh1, et1, er1, sem1))

        def start_gathers(cc, par):
            eh, et, er, sem = bufs[par]
            off = base + cc * C
            pltpu.sync_copy(idx_h.at[0, 0, pl.ds(off, C)], hi_v.at[par])
            pltpu.sync_copy(idx_h.at[1, 0, pl.ds(off, C)], ri_v.at[par])
            pltpu.sync_copy(idx_h.at[2, 0, pl.ds(off, C)], ti_v.at[par])
            return [
                pltpu.async_copy(ent_h.at[hi_v.at[par]], eh, sem),
                pltpu.async_copy(ent_h.at[ti_v.at[par]], et, sem),
                pltpu.async_copy(rel_h.at[ri_v.at[par]], er, sem),
            ]

        def compute(cc, par, cps):
            eh, et, er, _ = bufs[par]
            for cp in cps:
                cp.wait()

            @pl.loop(0, C)
            def _triple(i):
                acc = None
                for j in range(D // L):
                    re_sl = pl.ds(j * L, L)
                    im_sl = pl.ds(D + j * L, L)
                    hre = eh[i, re_sl]
                    him = eh[i, im_sl]
                    tre = et[i, re_sl]
                    tim = et[i, im_sl]
                    rre = er[i, re_sl]
                    rim = er[i, im_sl]
                    term = rre * (hre * tre + him * tim)
                    term = term + rim * (hre * tim - him * tre)
                    acc = term if acc is None else acc + term
                acc_v[i, :] = acc

            pltpu.sync_copy(acc_v, out_h.at[pl.ds(base + cc * C, C)])

        cps0 = start_gathers(0, 0)

        @pl.loop(0, NCHUNK, step=2)
        def _chunk(cc):
            cps1 = start_gathers(cc + 1, 1)
            compute(cc, 0, cps0)

            @pl.when(cc + 2 < NCHUNK)
            def _():
                start_gathers(cc + 2, 0)

            compute(cc + 1, 1, cps1)

    return kern(ent, rel, idx)


def _tc_reduce(partials):
    R = 2048

    def body(p_ref, n_ref, op_ref, on_ref):
        op_ref[...] = jnp.sum(p_ref[...], axis=1)
        on_ref[...] = jnp.sum(n_ref[...], axis=1)

    return pl.pallas_call(
        body,
        grid=(B // R,),
        in_specs=[pl.BlockSpec((R, L), lambda i: (i, 0)),
                  pl.BlockSpec((R, L), lambda i: (i + B // R, 0))],
        out_specs=[pl.BlockSpec((R,), lambda i: (i,)),
                   pl.BlockSpec((R,), lambda i: (i,))],
        out_shape=[jax.ShapeDtypeStruct((B,), jnp.float32)] * 2,
    )(partials, partials)


def kernel(pos_triples, neg_triples, ent_re, ent_im, rel_re, rel_im):
    ent, rel = _tc_transpose_concat(ent_re.T, ent_im.T, rel_re.T, rel_im.T)
    idx = jnp.concatenate([pos_triples.T, neg_triples.T], axis=1).reshape(3, 1, TB)
    partials = _sc_partials(ent, rel, idx)
    pos_scores, neg_scores = _tc_reduce(partials)
    return pos_scores, neg_scores


# speedup vs baseline: 1.0041x; 1.0041x over previous
import functools

import jax
import jax.numpy as jnp
from jax import lax
from jax.experimental import pallas as pl
from jax.experimental.pallas import tpu as pltpu
from jax.experimental.pallas import tpu_sc as plsc

B = 16384
TB = 2 * B
D = 64
W = 2 * D
L = 16
NC, NS = 2, 16
NW = NC * NS
BPW = TB // NW
C = 128
NCHUNK = BPW // C


def _tc_transpose_concat(ent_re_t, ent_im_t, rel_re_t, rel_im_t):
    N = ent_re_t.shape[1]
    EB = 8192

    def body(a_ref, b_ref, c_ref, d_ref, e_ref, r_ref):
        e_ref[:, :D] = a_ref[...].T
        e_ref[:, D:] = b_ref[...].T
        r_ref[:, :D] = c_ref[...].T
        r_ref[:, D:] = d_ref[...].T

    in_spec = pl.BlockSpec((D, EB), lambda i: (0, i))
    out_spec = pl.BlockSpec((EB, W), lambda i: (i, 0))
    return pl.pallas_call(
        body,
        grid=(pl.cdiv(N, EB),),
        in_specs=[in_spec] * 4,
        out_specs=[out_spec, out_spec],
        out_shape=[jax.ShapeDtypeStruct((N, W), jnp.float32)] * 2,
    )(ent_re_t, ent_im_t, rel_re_t, rel_im_t)


def _sc_partials(ent, rel, idx):
    mesh = plsc.VectorSubcoreMesh(core_axis_name="c", subcore_axis_name="s")

    @functools.partial(
        pl.kernel,
        mesh=mesh,
        out_type=jax.ShapeDtypeStruct((TB, L), jnp.float32),
        compiler_params=pltpu.CompilerParams(use_tc_tiling_on_sc=True),
        scratch_types=[
            pltpu.VMEM((2, C), jnp.int32),
            pltpu.VMEM((2, C), jnp.int32),
            pltpu.VMEM((2, C), jnp.int32),
            pltpu.VMEM((C, W), jnp.float32),
            pltpu.VMEM((C, W), jnp.float32),
            pltpu.VMEM((C, W), jnp.float32),
            pltpu.VMEM((C, W), jnp.float32),
            pltpu.VMEM((C, W), jnp.float32),
            pltpu.VMEM((C, W), jnp.float32),
            pltpu.VMEM((C, L), jnp.float32),
            pltpu.SemaphoreType.DMA,
            pltpu.SemaphoreType.DMA,
        ],
    )
    def kern(ent_h, rel_h, idx_h, out_h,
             hi_v, ri_v, ti_v, eh0, et0, er0, eh1, et1, er1,
             acc_v, sem0, sem1):
        wid = lax.axis_index("s") * NC + lax.axis_index("c")
        base = wid * BPW
        bufs = ((eh0, et0, er0, sem0), (eh1, et1, er1, sem1))

        def start_gathers(cc, par):
            eh, et, er, sem = bufs[par]
            off = base + cc * C
            pltpu.sync_copy(idx_h.at[0, 0, pl.ds(off, C)], hi_v.at[par])
            pltpu.sync_copy(idx_h.at[1, 0, pl.ds(off, C)], ri_v.at[par])
            pltpu.sync_copy(idx_h.at[2, 0, pl.ds(off, C)], ti_v.at[par])
            return [
                pltpu.async_copy(ent_h.at[hi_v.at[par]], eh, sem),
                pltpu.async_copy(ent_h.at[ti_v.at[par]], et, sem),
                pltpu.async_copy(rel_h.at[ri_v.at[par]], er, sem),
            ]

        def compute(cc, par, cps):
            eh, et, er, _ = bufs[par]
            for cp in cps:
                cp.wait()

            @pl.loop(0, C, step=2)
            def _triple(i):
                for u in range(2):
                    iu = i + u
                    acc = None
                    for j in range(D // L):
                        re_sl = pl.ds(j * L, L)
                        im_sl = pl.ds(D + j * L, L)
                        hre = eh[iu, re_sl]
                        him = eh[iu, im_sl]
                        tre = et[iu, re_sl]
                        tim = et[iu, im_sl]
                        rre = er[iu, re_sl]
                        rim = er[iu, im_sl]
                        term = rre * (hre * tre + him * tim)
                        term = term + rim * (hre * tim - him * tre)
                        acc = term if acc is None else acc + term
                    acc_v[iu, :] = acc

            pltpu.sync_copy(acc_v, out_h.at[pl.ds(base + cc * C, C)])

        cps0 = start_gathers(0, 0)

        @pl.loop(0, NCHUNK, step=2)
        def _chunk(cc):
            cps1 = start_gathers(cc + 1, 1)
            compute(cc, 0, cps0)

            @pl.when(cc + 2 < NCHUNK)
            def _():
                start_gathers(cc + 2, 0)

            compute(cc + 1, 1, cps1)

    return kern(ent, rel, idx)


def _tc_reduce(partials):
    R = 2048

    def body(p_ref, n_ref, op_ref, on_ref):
        op_ref[...] = jnp.sum(p_ref[...], axis=1)
        on_ref[...] = jnp.sum(n_ref[...], axis=1)

    return pl.pallas_call(
        body,
        grid=(B // R,),
        in_specs=[pl.BlockSpec((R, L), lambda i: (i, 0)),
                  pl.BlockSpec((R, L), lambda i: (i + B // R, 0))],
        out_specs=[pl.BlockSpec((R,), lambda i: (i,)),
                   pl.BlockSpec((R,), lambda i: (i,))],
        out_shape=[jax.ShapeDtypeStruct((B,), jnp.float32)] * 2,
    )(partials, partials)


def kernel(pos_triples, neg_triples, ent_re, ent_im, rel_re, rel_im):
    ent, rel = _tc_transpose_concat(ent_re.T, ent_im.T, rel_re.T, rel_im.T)
    idx = jnp.concatenate([pos_triples.T, neg_triples.T], axis=1).reshape(3, 1, TB)
    partials = _sc_partials(ent, rel, idx)
    pos_scores, neg_scores = _tc_reduce(partials)
    return pos_scores, neg_scores
